# Initial kernel scaffold; baseline (speedup 1.0000x reference)
import jax
import jax.numpy as jnp
from jax.experimental import pallas as pl


def kernel(x, emb):
    raise NotImplementedError("write your pallas kernel here")


# speedup vs baseline: 1.0166x; 1.0166x over previous
import functools

import jax
import jax.numpy as jnp
from jax import lax
from jax.experimental import pallas as pl
from jax.experimental.pallas import tpu as pltpu
from jax.experimental.pallas import tpu_sc as plsc

D = 256
K = 8192
B = 8
HW = 1024
N = B * HW
KT = 512
NUM_KT = K // KT


def _argmin_body(x2_ref, e2_ref, x_ref, emb_ref, idx_ref, minv_ref,
                 run_min, run_idx):
    k = pl.program_id(0)
    embt = emb_ref[...]
    e2t = e2_ref[0, :, :]

    for b in range(B):
        xb = x_ref[b]
        s = jnp.dot(embt, xb, preferred_element_type=jnp.float32)
        d = (x2_ref[pl.ds(b, 1), :] + e2t) - 2.0 * s
        mv = jnp.min(d, axis=0, keepdims=True)
        rows = lax.broadcasted_iota(jnp.int32, d.shape, 0)
        mi = jnp.min(jnp.where(d == mv, rows, KT), axis=0,
                     keepdims=True) + k * KT

        @pl.when(k == 0)
        def _():
            run_min[pl.ds(b, 1), :] = mv
            run_idx[pl.ds(b, 1), :] = mi

        @pl.when(k > 0)
        def _():
            prev_min = run_min[pl.ds(b, 1), :]
            prev_idx = run_idx[pl.ds(b, 1), :]
            better = mv < prev_min
            run_min[pl.ds(b, 1), :] = jnp.where(better, mv, prev_min)
            run_idx[pl.ds(b, 1), :] = jnp.where(better, mi, prev_idx)

    @pl.when(k == NUM_KT - 1)
    def _():
        idx_ref[...] = run_idx[...]
        minv_ref[...] = run_min[...]


_argmin_call = pl.pallas_call(
    _argmin_body,
    grid=(NUM_KT,),
    in_specs=[
        pl.BlockSpec((B, HW), lambda k: (0, 0)),
        pl.BlockSpec((1, KT, 1), lambda k: (k, 0, 0)),
        pl.BlockSpec((B, D, HW), lambda k: (0, 0, 0)),
        pl.BlockSpec((KT, D), lambda k: (k, 0)),
    ],
    out_specs=[
        pl.BlockSpec((B, HW), lambda k: (0, 0)),
        pl.BlockSpec((B, HW), lambda k: (0, 0)),
    ],
    out_shape=[
        jax.ShapeDtypeStruct((B, HW), jnp.int32),
        jax.ShapeDtypeStruct((B, HW), jnp.float32),
    ],
    scratch_shapes=[
        pltpu.VMEM((B, HW), jnp.float32),
        pltpu.VMEM((B, HW), jnp.int32),
    ],
)


def _gather(emb, idx):
    info = plsc.get_sparse_core_info()
    nw = info.num_cores * info.num_subcores
    bpw = N // nw
    mesh = plsc.VectorSubcoreMesh(core_axis_name="c", subcore_axis_name="s")

    @functools.partial(
        pl.kernel, mesh=mesh,
        out_type=jax.ShapeDtypeStruct((N, D), jnp.float32),
        scratch_types=[
            pltpu.VMEM((bpw,), jnp.int32),
            pltpu.VMEM((bpw, D), jnp.float32),
            pltpu.SemaphoreType.DMA,
        ],
    )
    def sc_gather(table_hbm, idx_hbm, out_hbm, idx_v, rows_v, sem):
        wid = lax.axis_index("s") * info.num_cores + lax.axis_index("c")
        base = wid * bpw
        pltpu.sync_copy(idx_hbm.at[pl.ds(base, bpw)], idx_v)
        pltpu.async_copy(table_hbm.at[idx_v], rows_v, sem).wait()
        pltpu.sync_copy(rows_v, out_hbm.at[pl.ds(base, bpw)])

    return sc_gather(emb, idx)


def kernel(x, emb):
    xp = jnp.transpose(x, (0, 2, 3, 1))
    flat_x = xp.reshape(-1, D)
    x2 = jnp.sum(flat_x ** 2, axis=1).reshape(B, HW)
    e2 = jnp.sum(emb ** 2, axis=1).reshape(NUM_KT, KT, 1)
    x3 = x.reshape(B, D, HW)

    idx, minv = _argmin_call(x2, e2, x3, emb)

    q = _gather(emb, idx.reshape(N))

    quantized = q.reshape(B, HW, D).transpose(0, 2, 1).reshape(x.shape)
    loss = 1.25 * jnp.sum(minv) / (N * D)
    return (quantized, loss)


# speedup vs baseline: 1.1907x; 1.1712x over previous
import functools

import jax
import jax.numpy as jnp
from jax import lax
from jax.experimental import pallas as pl
from jax.experimental.pallas import tpu as pltpu
from jax.experimental.pallas import tpu_sc as plsc

D = 256
K = 8192
B = 8
HW = 1024
N = B * HW
KT = 512
NUM_KT = K // KT


def _argmin_body(x2_ref, e2_ref, x_ref, emb_ref, idx_ref, minv_ref,
                 run_min, run_idx):
    k = pl.program_id(0)
    embt = emb_ref[...]
    e2t = e2_ref[0, :, :]

    for b in range(B):
        xb = x_ref[b]
        s = jnp.dot(embt, xb, preferred_element_type=jnp.float32)
        d = (x2_ref[pl.ds(b, 1), :] + e2t) - s
        mv = jnp.min(d, axis=0, keepdims=True)
        mi = (jnp.argmin(d, axis=0).astype(jnp.int32)[None, :]
              + k * KT)

        @pl.when(k == 0)
        def _():
            run_min[pl.ds(b, 1), :] = mv
            run_idx[pl.ds(b, 1), :] = mi

        @pl.when(k > 0)
        def _():
            prev_min = run_min[pl.ds(b, 1), :]
            prev_idx = run_idx[pl.ds(b, 1), :]
            better = mv < prev_min
            run_min[pl.ds(b, 1), :] = jnp.where(better, mv, prev_min)
            run_idx[pl.ds(b, 1), :] = jnp.where(better, mi, prev_idx)

    @pl.when(k == NUM_KT - 1)
    def _():
        idx_ref[...] = run_idx[...]
        minv_ref[...] = run_min[...]


_argmin_call = pl.pallas_call(
    _argmin_body,
    grid=(NUM_KT,),
    in_specs=[
        pl.BlockSpec((B, HW), lambda k: (0, 0)),
        pl.BlockSpec((1, KT, 1), lambda k: (k, 0, 0)),
        pl.BlockSpec((B, D, HW), lambda k: (0, 0, 0)),
        pl.BlockSpec((KT, D), lambda k: (k, 0)),
    ],
    out_specs=[
        pl.BlockSpec((B, HW), lambda k: (0, 0)),
        pl.BlockSpec((B, HW), lambda k: (0, 0)),
    ],
    out_shape=[
        jax.ShapeDtypeStruct((B, HW), jnp.int32),
        jax.ShapeDtypeStruct((B, HW), jnp.float32),
    ],
    scratch_shapes=[
        pltpu.VMEM((B, HW), jnp.float32),
        pltpu.VMEM((B, HW), jnp.int32),
    ],
)


def _gather(emb, idx):
    info = plsc.get_sparse_core_info()
    nw = info.num_cores * info.num_subcores
    bpw = N // nw
    mesh = plsc.VectorSubcoreMesh(core_axis_name="c", subcore_axis_name="s")

    @functools.partial(
        pl.kernel, mesh=mesh,
        out_type=jax.ShapeDtypeStruct((N, D), jnp.float32),
        scratch_types=[
            pltpu.VMEM((bpw,), jnp.int32),
            pltpu.VMEM((bpw, D), jnp.float32),
            pltpu.SemaphoreType.DMA,
        ],
    )
    def sc_gather(table_hbm, idx_hbm, out_hbm, idx_v, rows_v, sem):
        wid = lax.axis_index("s") * info.num_cores + lax.axis_index("c")
        base = wid * bpw
        pltpu.sync_copy(idx_hbm.at[pl.ds(base, bpw)], idx_v)
        pltpu.async_copy(table_hbm.at[idx_v], rows_v, sem).wait()
        pltpu.sync_copy(rows_v, out_hbm.at[pl.ds(base, bpw)])

    return sc_gather(emb, idx)


def kernel(x, emb):
    xp = jnp.transpose(x, (0, 2, 3, 1))
    flat_x = xp.reshape(-1, D)
    x2 = (0.5 * jnp.sum(flat_x ** 2, axis=1)).reshape(B, HW)
    e2 = (0.5 * jnp.sum(emb ** 2, axis=1)).reshape(NUM_KT, KT, 1)
    x3 = x.reshape(B, D, HW)

    idx, minv = _argmin_call(x2, e2, x3, emb)

    q = _gather(emb, idx.reshape(N))

    quantized = q.reshape(B, HW, D).transpose(0, 2, 1).reshape(x.shape)
    loss = 2.5 * jnp.sum(minv) / (N * D)
    return (quantized, loss)


# speedup vs baseline: 1.4405x; 1.2098x over previous
import functools

import jax
import jax.numpy as jnp
from jax import lax
from jax.experimental import pallas as pl
from jax.experimental.pallas import tpu as pltpu
from jax.experimental.pallas import tpu_sc as plsc

D = 256
K = 8192
B = 8
HW = 1024
N = B * HW
KT = 4096
NUM_KT = K // KT


def _argmin_body(x2_ref, e2_ref, x_ref, emb_ref, idx_ref, loss_ref,
                 run_min, run_idx):
    k = pl.program_id(0)
    embt = emb_ref[...]
    e2t = e2_ref[0, :, :]

    for b in range(B):
        xb = x_ref[b]
        s = jnp.dot(embt, xb, preferred_element_type=jnp.float32)
        d = (x2_ref[pl.ds(b, 1), :] + e2t) - s
        mv = jnp.min(d, axis=0, keepdims=True)
        mi = (jnp.argmin(d, axis=0).astype(jnp.int32)[None, :]
              + k * KT)

        @pl.when(k == 0)
        def _():
            run_min[pl.ds(b, 1), :] = mv
            run_idx[pl.ds(b, 1), :] = mi

        @pl.when(k > 0)
        def _():
            prev_min = run_min[pl.ds(b, 1), :]
            prev_idx = run_idx[pl.ds(b, 1), :]
            better = mv < prev_min
            run_min[pl.ds(b, 1), :] = jnp.where(better, mv, prev_min)
            run_idx[pl.ds(b, 1), :] = jnp.where(better, mi, prev_idx)

    @pl.when(k == NUM_KT - 1)
    def _():
        idx_ref[...] = run_idx[...]
        loss_ref[0:1, 0:1] = jnp.sum(run_min[...], keepdims=True)


_argmin_call = pl.pallas_call(
    _argmin_body,
    grid=(NUM_KT,),
    in_specs=[
        pl.BlockSpec((B, HW), lambda k: (0, 0)),
        pl.BlockSpec((1, KT, 1), lambda k: (k, 0, 0)),
        pl.BlockSpec((B, D, HW), lambda k: (0, 0, 0)),
        pl.BlockSpec((KT, D), lambda k: (k, 0)),
    ],
    out_specs=[
        pl.BlockSpec((B, HW), lambda k: (0, 0)),
        pl.BlockSpec((1, 1), lambda k: (0, 0)),
    ],
    out_shape=[
        jax.ShapeDtypeStruct((B, HW), jnp.int32),
        jax.ShapeDtypeStruct((1, 1), jnp.float32),
    ],
    scratch_shapes=[
        pltpu.VMEM((B, HW), jnp.float32),
        pltpu.VMEM((B, HW), jnp.int32),
    ],
)


def _gather(emb, idx):
    info = plsc.get_sparse_core_info()
    nw = info.num_cores * info.num_subcores
    bpw = N // nw
    mesh = plsc.VectorSubcoreMesh(core_axis_name="c", subcore_axis_name="s")

    @functools.partial(
        pl.kernel, mesh=mesh,
        out_type=jax.ShapeDtypeStruct((N, D), jnp.float32),
        scratch_types=[
            pltpu.VMEM((bpw,), jnp.int32),
            pltpu.VMEM((bpw, D), jnp.float32),
            pltpu.SemaphoreType.DMA,
        ],
    )
    def sc_gather(table_hbm, idx_hbm, out_hbm, idx_v, rows_v, sem):
        wid = lax.axis_index("s") * info.num_cores + lax.axis_index("c")
        base = wid * bpw
        pltpu.sync_copy(idx_hbm.at[pl.ds(base, bpw)], idx_v)
        pltpu.async_copy(table_hbm.at[idx_v], rows_v, sem).wait()
        pltpu.sync_copy(rows_v, out_hbm.at[pl.ds(base, bpw)])

    return sc_gather(emb, idx)


def kernel(x, emb):
    xp = jnp.transpose(x, (0, 2, 3, 1))
    flat_x = xp.reshape(-1, D)
    x2 = (0.5 * jnp.sum(flat_x ** 2, axis=1)).reshape(B, HW)
    e2 = (0.5 * jnp.sum(emb ** 2, axis=1)).reshape(NUM_KT, KT, 1)
    x3 = x.reshape(B, D, HW)

    idx, losssum = _argmin_call(x2, e2, x3, emb)

    q = _gather(emb, idx.reshape(N))

    quantized = q.reshape(B, HW, D).transpose(0, 2, 1).reshape(x.shape)
    loss = 2.5 * losssum[0, 0] / (N * D)
    return (quantized, loss)


# speedup vs baseline: 1.5353x; 1.0659x over previous
import functools

import jax
import jax.numpy as jnp
from jax import lax
from jax.experimental import pallas as pl
from jax.experimental.pallas import tpu as pltpu
from jax.experimental.pallas import tpu_sc as plsc

D = 256
K = 8192
B = 8
HW = 1024
N = B * HW
KT = 4096
NUM_KT = K // KT


def _argmin_body(x2_ref, x_ref, emb_ref, idx_ref, loss_ref,
                 run_min, run_idx):
    k = pl.program_id(0)
    embt = emb_ref[...]
    e2t = 0.5 * jnp.sum(embt * embt, axis=1, keepdims=True)

    for b in range(B):
        xb = x_ref[b]
        s = jnp.dot(embt, xb, preferred_element_type=jnp.float32)
        d = (x2_ref[pl.ds(b, 1), :] + e2t) - s
        mv = jnp.min(d, axis=0, keepdims=True)
        mi = (jnp.argmin(d, axis=0).astype(jnp.int32)[None, :]
              + k * KT)

        @pl.when(k == 0)
        def _():
            run_min[pl.ds(b, 1), :] = mv
            run_idx[pl.ds(b, 1), :] = mi

        @pl.when(k > 0)
        def _():
            prev_min = run_min[pl.ds(b, 1), :]
            prev_idx = run_idx[pl.ds(b, 1), :]
            better = mv < prev_min
            run_min[pl.ds(b, 1), :] = jnp.where(better, mv, prev_min)
            run_idx[pl.ds(b, 1), :] = jnp.where(better, mi, prev_idx)

    @pl.when(k == NUM_KT - 1)
    def _():
        idx_ref[...] = run_idx[...]
        loss_ref[0:1, 0:1] = jnp.sum(run_min[...], keepdims=True)


_argmin_call = pl.pallas_call(
    _argmin_body,
    grid=(NUM_KT,),
    in_specs=[
        pl.BlockSpec((B, HW), lambda k: (0, 0)),
        pl.BlockSpec((B, D, HW), lambda k: (0, 0, 0)),
        pl.BlockSpec((KT, D), lambda k: (k, 0)),
    ],
    out_specs=[
        pl.BlockSpec((B, HW), lambda k: (0, 0)),
        pl.BlockSpec((1, 1), lambda k: (0, 0)),
    ],
    out_shape=[
        jax.ShapeDtypeStruct((B, HW), jnp.int32),
        jax.ShapeDtypeStruct((1, 1), jnp.float32),
    ],
    scratch_shapes=[
        pltpu.VMEM((B, HW), jnp.float32),
        pltpu.VMEM((B, HW), jnp.int32),
    ],
)


def _gather(emb, idx):
    info = plsc.get_sparse_core_info()
    nw = info.num_cores * info.num_subcores
    bpw = N // nw
    half = bpw // 2
    mesh = plsc.VectorSubcoreMesh(core_axis_name="c", subcore_axis_name="s")

    @functools.partial(
        pl.kernel, mesh=mesh,
        out_type=jax.ShapeDtypeStruct((N, D), jnp.float32),
        scratch_types=[
            pltpu.VMEM((half,), jnp.int32),
            pltpu.VMEM((half,), jnp.int32),
            pltpu.VMEM((half, D), jnp.float32),
            pltpu.VMEM((half, D), jnp.float32),
            pltpu.SemaphoreType.DMA,
            pltpu.SemaphoreType.DMA,
            pltpu.SemaphoreType.DMA,
            pltpu.SemaphoreType.DMA,
        ],
    )
    def sc_gather(table_hbm, idx_hbm, out_hbm, idx0, idx1, rows0, rows1,
                  g0s, g1s, w0s, w1s):
        wid = lax.axis_index("s") * info.num_cores + lax.axis_index("c")
        base = wid * bpw
        pltpu.sync_copy(idx_hbm.at[pl.ds(base, half)], idx0)
        pltpu.sync_copy(idx_hbm.at[pl.ds(base + half, half)], idx1)
        g0 = pltpu.async_copy(table_hbm.at[idx0], rows0, g0s)
        g1 = pltpu.async_copy(table_hbm.at[idx1], rows1, g1s)
        g0.wait()
        w0 = pltpu.async_copy(rows0, out_hbm.at[pl.ds(base, half)], w0s)
        g1.wait()
        w1 = pltpu.async_copy(rows1, out_hbm.at[pl.ds(base + half, half)], w1s)
        w0.wait()
        w1.wait()

    return sc_gather(emb, idx)


def kernel(x, emb):
    xp = jnp.transpose(x, (0, 2, 3, 1))
    flat_x = xp.reshape(-1, D)
    x2 = (0.5 * jnp.sum(flat_x ** 2, axis=1)).reshape(B, HW)
    x3 = x.reshape(B, D, HW)

    idx, losssum = _argmin_call(x2, x3, emb)

    q = _gather(emb, idx.reshape(N))

    quantized = q.reshape(B, HW, D).transpose(0, 2, 1).reshape(x.shape)
    loss = 2.5 * losssum[0, 0] / (N * D)
    return (quantized, loss)


# speedup vs baseline: 1.5597x; 1.0159x over previous
import functools

import jax
import jax.numpy as jnp
from jax import lax
from jax.experimental import pallas as pl
from jax.experimental.pallas import tpu as pltpu
from jax.experimental.pallas import tpu_sc as plsc

D = 256
K = 8192
B = 8
HW = 1024
N = B * HW
KT = 4096
NUM_KT = K // KT


def _argmin_body(x_ref, emb_ref, idx_ref, loss_ref,
                 run_min, run_idx, x2s):
    k = pl.program_id(0)
    embt = emb_ref[...]
    e2t = 0.5 * jnp.sum(embt * embt, axis=1, keepdims=True)

    for b in range(B):
        xb = x_ref[b]

        @pl.when(k == 0)
        def _():
            x2s[pl.ds(b, 1), :] = 0.5 * jnp.sum(xb * xb, axis=0,
                                                keepdims=True)

        s = jnp.dot(embt, xb, preferred_element_type=jnp.float32)
        d = (x2s[pl.ds(b, 1), :] + e2t) - s
        mv = jnp.min(d, axis=0, keepdims=True)
        mi = (jnp.argmin(d, axis=0).astype(jnp.int32)[None, :]
              + k * KT)

        @pl.when(k == 0)
        def _():
            run_min[pl.ds(b, 1), :] = mv
            run_idx[pl.ds(b, 1), :] = mi

        @pl.when(k > 0)
        def _():
            prev_min = run_min[pl.ds(b, 1), :]
            prev_idx = run_idx[pl.ds(b, 1), :]
            better = mv < prev_min
            run_min[pl.ds(b, 1), :] = jnp.where(better, mv, prev_min)
            run_idx[pl.ds(b, 1), :] = jnp.where(better, mi, prev_idx)

    @pl.when(k == NUM_KT - 1)
    def _():
        idx_ref[...] = run_idx[...]
        loss_ref[0:1, 0:1] = jnp.sum(run_min[...], keepdims=True)


_argmin_call = pl.pallas_call(
    _argmin_body,
    grid=(NUM_KT,),
    in_specs=[
        pl.BlockSpec((B, D, HW), lambda k: (0, 0, 0)),
        pl.BlockSpec((KT, D), lambda k: (k, 0)),
    ],
    out_specs=[
        pl.BlockSpec((B, HW), lambda k: (0, 0)),
        pl.BlockSpec((1, 1), lambda k: (0, 0)),
    ],
    out_shape=[
        jax.ShapeDtypeStruct((B, HW), jnp.int32),
        jax.ShapeDtypeStruct((1, 1), jnp.float32),
    ],
    scratch_shapes=[
        pltpu.VMEM((B, HW), jnp.float32),
        pltpu.VMEM((B, HW), jnp.int32),
        pltpu.VMEM((B, HW), jnp.float32),
    ],
)


def _gather(emb, idx):
    info = plsc.get_sparse_core_info()
    nw = info.num_cores * info.num_subcores
    bpw = N // nw
    half = bpw // 2
    mesh = plsc.VectorSubcoreMesh(core_axis_name="c", subcore_axis_name="s")

    @functools.partial(
        pl.kernel, mesh=mesh,
        out_type=jax.ShapeDtypeStruct((N, D), jnp.float32),
        scratch_types=[
            pltpu.VMEM((half,), jnp.int32),
            pltpu.VMEM((half,), jnp.int32),
            pltpu.VMEM((half, D), jnp.float32),
            pltpu.VMEM((half, D), jnp.float32),
            pltpu.SemaphoreType.DMA,
            pltpu.SemaphoreType.DMA,
            pltpu.SemaphoreType.DMA,
            pltpu.SemaphoreType.DMA,
        ],
    )
    def sc_gather(table_hbm, idx_hbm, out_hbm, idx0, idx1, rows0, rows1,
                  g0s, g1s, w0s, w1s):
        wid = lax.axis_index("s") * info.num_cores + lax.axis_index("c")
        base = wid * bpw
        pltpu.sync_copy(idx_hbm.at[pl.ds(base, half)], idx0)
        pltpu.sync_copy(idx_hbm.at[pl.ds(base + half, half)], idx1)
        g0 = pltpu.async_copy(table_hbm.at[idx0], rows0, g0s)
        g1 = pltpu.async_copy(table_hbm.at[idx1], rows1, g1s)
        g0.wait()
        w0 = pltpu.async_copy(rows0, out_hbm.at[pl.ds(base, half)], w0s)
        g1.wait()
        w1 = pltpu.async_copy(rows1, out_hbm.at[pl.ds(base + half, half)], w1s)
        w0.wait()
        w1.wait()

    return sc_gather(emb, idx)


def kernel(x, emb):
    x3 = x.reshape(B, D, HW)

    idx, losssum = _argmin_call(x3, emb)

    q = _gather(emb, idx.reshape(N))

    quantized = q.reshape(B, HW, D).transpose(0, 2, 1).reshape(x.shape)
    loss = 2.5 * losssum[0, 0] / (N * D)
    return (quantized, loss)
